# Initial kernel scaffold; baseline (speedup 1.0000x reference)
import jax
import jax.numpy as jnp
from jax.experimental import pallas as pl


def kernel(x, edge_index, h, W, V, alpha):
    raise NotImplementedError("write your pallas kernel here")


# speedup vs baseline: 2.2712x; 2.2712x over previous
import functools

import jax
import jax.numpy as jnp
from jax import lax
from jax.experimental import pallas as pl
from jax.experimental.pallas import tpu as pltpu
from jax.experimental.pallas import tpu_sc as plsc

NODES = 10000
DIM = 128
OUT = 128

NC = 2
NS = 16
NW = NC * NS
CH = 128
NCH = 320
E_PAD = NW * NCH * CH
NODES_PAD = 10112
ZROWS = NODES_PAD // NS


def _sc_segment_sum(h, src, dst, zeros):
    mesh = plsc.VectorSubcoreMesh(core_axis_name="c", subcore_axis_name="s")

    @functools.partial(
        pl.kernel,
        mesh=mesh,
        out_type=jax.ShapeDtypeStruct((NC, NODES_PAD, OUT), jnp.float32),
        scratch_types=[
            pltpu.VMEM_SHARED((NODES_PAD, OUT), jnp.float32),
            pltpu.VMEM((CH,), jnp.int32),
            pltpu.VMEM((CH,), jnp.int32),
            pltpu.VMEM((CH, OUT), jnp.float32),
            pltpu.SemaphoreType.DMA,
        ],
    )
    def k(h_hbm, src_hbm, dst_hbm, zeros_hbm, out_hbm,
          acc, src_v, dst_v, rows_v, sem):
        cid = lax.axis_index("c")
        sid = lax.axis_index("s")
        wid = cid * NS + sid

        pltpu.sync_copy(zeros_hbm.at[pl.ds(sid * ZROWS, ZROWS)],
                        acc.at[pl.ds(sid * ZROWS, ZROWS)])
        plsc.subcore_barrier()

        def body(ci, carry):
            pltpu.sync_copy(src_hbm.at[wid, ci], src_v)
            pltpu.sync_copy(dst_hbm.at[wid, ci], dst_v)
            pltpu.async_copy(h_hbm.at[src_v], rows_v, sem).wait()
            pltpu.sync_copy(rows_v, acc.at[dst_v], add=True)
            return carry

        lax.fori_loop(0, NCH, body, 0)
        plsc.subcore_barrier()

        pltpu.sync_copy(acc.at[pl.ds(sid * ZROWS, ZROWS)],
                        out_hbm.at[cid, pl.ds(sid * ZROWS, ZROWS)])

    return k(h, src, dst, zeros)


def _tc_finish_body(x_ref, w_ref, p_ref, v_ref, o_ref):
    xw = jnp.dot(x_ref[...], w_ref[...], preferred_element_type=jnp.float32)
    s = p_ref[0] + p_ref[1]
    sv = jnp.dot(s, v_ref[...], preferred_element_type=jnp.float32)
    o_ref[...] = jnp.maximum(xw + sv, 0.0)


def _tc_finish(x, W, partials, V):
    BM = 2000
    grid = (NODES // BM,)
    return pl.pallas_call(
        _tc_finish_body,
        grid=grid,
        in_specs=[
            pl.BlockSpec((BM, DIM), lambda i: (i, 0)),
            pl.BlockSpec((DIM, OUT), lambda i: (0, 0)),
            pl.BlockSpec((NC, BM, OUT), lambda i: (0, i, 0)),
            pl.BlockSpec((OUT, OUT), lambda i: (0, 0)),
        ],
        out_specs=pl.BlockSpec((BM, OUT), lambda i: (i, 0)),
        out_shape=jax.ShapeDtypeStruct((NODES, OUT), jnp.float32),
    )(x, W, partials, V)


def kernel(x, edge_index, h, W, V, alpha):
    del alpha
    src = edge_index[:, 0, :].reshape(-1).astype(jnp.int32)
    dst = edge_index[:, 1, :].reshape(-1).astype(jnp.int32)
    pad = E_PAD - src.shape[0]
    src = jnp.concatenate([src, jnp.zeros((pad,), jnp.int32)])
    dst = jnp.concatenate([dst, jnp.full((pad,), NODES, jnp.int32)])
    src = src.reshape(NW, NCH, CH)
    dst = dst.reshape(NW, NCH, CH)
    zeros = jnp.zeros((NODES_PAD, OUT), jnp.float32)
    partials = _sc_segment_sum(h, src, dst, zeros)
    return _tc_finish(x, W, partials, V)


# speedup vs baseline: 2.8904x; 1.2726x over previous
import functools

import jax
import jax.numpy as jnp
from jax import lax
from jax.experimental import pallas as pl
from jax.experimental.pallas import tpu as pltpu
from jax.experimental.pallas import tpu_sc as plsc

NODES = 10000
DIM = 128
OUT = 128

NC = 2
NS = 16
NW = NC * NS
CH = 128
NCH = 320
E_PAD = NW * NCH * CH
NODES_PAD = 10112
ZROWS = NODES_PAD // NS


def _sc_segment_sum(h, src, dst, zeros):
    mesh = plsc.VectorSubcoreMesh(core_axis_name="c", subcore_axis_name="s")
    NBUF = 2
    SUP = 32
    NSUP = NCH // SUP

    @functools.partial(
        pl.kernel,
        mesh=mesh,
        out_type=jax.ShapeDtypeStruct((NC, NODES_PAD, OUT), jnp.float32),
        scratch_types=[
            pltpu.VMEM_SHARED((NODES_PAD, OUT), jnp.float32),
            pltpu.VMEM((SUP, CH), jnp.int32),
            pltpu.VMEM((SUP, CH), jnp.int32),
            pltpu.VMEM((NBUF, CH, OUT), jnp.float32),
            pltpu.SemaphoreType.DMA,
            pltpu.SemaphoreType.DMA,
        ],
    )
    def k(h_hbm, src_hbm, dst_hbm, zeros_hbm, out_hbm,
          acc, src_v, dst_v, rows_v, sem0, sem1):
        sems = [sem0, sem1]
        cid = lax.axis_index("c")
        sid = lax.axis_index("s")
        wid = cid * NS + sid

        pltpu.sync_copy(zeros_hbm.at[pl.ds(sid * ZROWS, ZROWS)],
                        acc.at[pl.ds(sid * ZROWS, ZROWS)])
        plsc.subcore_barrier()

        def sup_body(sp, carry):
            pltpu.sync_copy(src_hbm.at[wid, pl.ds(sp * SUP, SUP)], src_v)
            pltpu.sync_copy(dst_hbm.at[wid, pl.ds(sp * SUP, SUP)], dst_v)
            for b in range(NBUF):
                pltpu.async_copy(h_hbm.at[src_v.at[b]], rows_v.at[b], sems[b])

            def body(g, carry2):
                base = g * NBUF
                for b in range(NBUF):
                    c = base + b
                    pltpu.make_async_copy(
                        h_hbm.at[src_v.at[c]], rows_v.at[b], sems[b]).wait()
                    pltpu.sync_copy(rows_v.at[b], acc.at[dst_v.at[c]],
                                    add=True)

                    @pl.when(c + NBUF < SUP)
                    def _():
                        pltpu.async_copy(
                            h_hbm.at[src_v.at[c + NBUF]], rows_v.at[b],
                            sems[b])
                return carry2

            lax.fori_loop(0, SUP // NBUF, body, 0)
            return carry

        lax.fori_loop(0, NSUP, sup_body, 0)
        plsc.subcore_barrier()

        pltpu.sync_copy(acc.at[pl.ds(sid * ZROWS, ZROWS)],
                        out_hbm.at[cid, pl.ds(sid * ZROWS, ZROWS)])

    return k(h, src, dst, zeros)


def _tc_finish_body(x_ref, w_ref, p_ref, v_ref, o_ref):
    xw = jnp.dot(x_ref[...], w_ref[...], preferred_element_type=jnp.float32)
    s = p_ref[0] + p_ref[1]
    sv = jnp.dot(s, v_ref[...], preferred_element_type=jnp.float32)
    o_ref[...] = jnp.maximum(xw + sv, 0.0)


def _tc_finish(x, W, partials, V):
    BM = 2000
    grid = (NODES // BM,)
    return pl.pallas_call(
        _tc_finish_body,
        grid=grid,
        in_specs=[
            pl.BlockSpec((BM, DIM), lambda i: (i, 0)),
            pl.BlockSpec((DIM, OUT), lambda i: (0, 0)),
            pl.BlockSpec((NC, BM, OUT), lambda i: (0, i, 0)),
            pl.BlockSpec((OUT, OUT), lambda i: (0, 0)),
        ],
        out_specs=pl.BlockSpec((BM, OUT), lambda i: (i, 0)),
        out_shape=jax.ShapeDtypeStruct((NODES, OUT), jnp.float32),
    )(x, W, partials, V)


def kernel(x, edge_index, h, W, V, alpha):
    del alpha
    src = edge_index[:, 0, :].reshape(-1).astype(jnp.int32)
    dst = edge_index[:, 1, :].reshape(-1).astype(jnp.int32)
    pad = E_PAD - src.shape[0]
    src = jnp.concatenate([src, jnp.zeros((pad,), jnp.int32)])
    dst = jnp.concatenate([dst, jnp.full((pad,), NODES, jnp.int32)])
    src = src.reshape(NW, NCH, CH)
    dst = dst.reshape(NW, NCH, CH)
    zeros = jnp.zeros((NODES_PAD, OUT), jnp.float32)
    partials = _sc_segment_sum(h, src, dst, zeros)
    return _tc_finish(x, W, partials, V)
